# baseline (speedup 1.0000x reference)
import functools
import jax
import jax.numpy as jnp
from jax import lax
from jax.experimental import pallas as pl
from jax.experimental.pallas import tpu as pltpu
from jax.experimental.pallas import tpu_sc as plsc

BATCH = 16
BEAM = 8
VOCAB = 100000
HIDDEN = 512
MAX_LEN = 8
SOS_ID = 1
EOS_ID = 2
NEG_INF = -1e9
BK = BATCH * BEAM

TILE = 2048
NT = (VOCAB + TILE - 1) // TILE
BIG_NEG = -1e30


_N_WORKERS = 16
_ROWS_PER_W = BK // _N_WORKERS


def _sc_gather_body(table_hbm, idx_hbm, out_hbm, idx_v, rows_v, sem):
    wid = lax.axis_index("s") * 2 + lax.axis_index("c")

    @pl.when(wid < _N_WORKERS)
    def _():
        base = wid * _ROWS_PER_W
        pltpu.sync_copy(idx_hbm.at[pl.ds(base, _ROWS_PER_W)], idx_v)
        pltpu.async_copy(table_hbm.at[idx_v], rows_v, sem).wait()
        pltpu.sync_copy(rows_v, out_hbm.at[pl.ds(base, _ROWS_PER_W)])


def _sc_gather(embedding, idx):
    mesh = plsc.VectorSubcoreMesh(core_axis_name="c", subcore_axis_name="s")
    k = functools.partial(
        pl.kernel, mesh=mesh,
        out_type=jax.ShapeDtypeStruct((BK, HIDDEN), jnp.float32),
        scratch_types=[
            pltpu.VMEM((_ROWS_PER_W,), jnp.int32),
            pltpu.VMEM((_ROWS_PER_W, HIDDEN), jnp.float32),
            pltpu.SemaphoreType.DMA,
        ],
    )(_sc_gather_body)
    return k(embedding, idx)


def _step_kernel(emb_ref, hprev_ref, wih_ref, whh_ref, wout_ref,
                 hout_ref, v8_ref, i8_ref, s_ref,
                 h_scr, v8_scr, i8_scr, s_scr):
    j = pl.program_id(0)

    @pl.when(j == 0)
    def _init():
        emb = emb_ref[...]
        h = jnp.tanh(
            jax.lax.dot_general(emb, wih_ref[...], (((1,), (0,)), ((), ())),
                                preferred_element_type=jnp.float32)
            + jax.lax.dot_general(hprev_ref[...], whh_ref[...],
                                  (((1,), (0,)), ((), ())),
                                  preferred_element_type=jnp.float32))
        h_scr[...] = h
        hout_ref[...] = h
        v8_scr[...] = jnp.full((BK, 8), BIG_NEG, dtype=jnp.float32)
        i8_scr[...] = jnp.zeros((BK, 8), dtype=jnp.int32)
        s_scr[...] = jnp.zeros((BK, 8), dtype=jnp.float32)

    h = h_scr[...]
    logits = jax.lax.dot_general(h, wout_ref[...], (((1,), (0,)), ((), ())),
                                 preferred_element_type=jnp.float32)
    col = jax.lax.broadcasted_iota(jnp.int32, (BK, TILE), 1)
    logits = jnp.where(col < VOCAB - j * TILE, logits, BIG_NEG)
    s = s_scr[:, 0:1] + jnp.sum(jnp.exp(logits), axis=1, keepdims=True)
    s_scr[...] = jnp.broadcast_to(s, (BK, 8))
    tmax = jnp.max(logits, axis=1, keepdims=True)

    v8 = v8_scr[...]
    i8 = i8_scr[...]
    slot_iota = jax.lax.broadcasted_iota(jnp.int32, (BK, 8), 1)

    def cond(carry):
        L, m, v8c, i8c, k = carry
        return jnp.logical_and(
            k < 8, jnp.any(m > jnp.min(v8c, axis=1, keepdims=True)))

    def body(carry):
        L, m, v8c, i8c, k = carry
        pos = jnp.min(jnp.where(L == m, col, TILE), axis=1, keepdims=True)
        minv = jnp.min(v8c, axis=1, keepdims=True)
        improve = m > minv
        is_min = v8c == minv
        si = jnp.min(jnp.where(is_min, slot_iota, 8), axis=1, keepdims=True)
        repl = jnp.logical_and(slot_iota == si, improve)
        v8c = jnp.where(repl, jnp.broadcast_to(m, (BK, 8)), v8c)
        i8c = jnp.where(repl, jnp.broadcast_to(pos + j * TILE, (BK, 8)), i8c)
        L = jnp.where(jnp.logical_and(col == pos, improve), BIG_NEG, L)
        m2 = jnp.max(L, axis=1, keepdims=True)
        return (L, m2, v8c, i8c, k + 1)

    init = (logits, tmax, v8, i8, jnp.int32(0))
    _, _, v8, i8, _ = lax.while_loop(cond, body, init)
    v8_scr[...] = v8
    i8_scr[...] = i8

    @pl.when(j == NT - 1)
    def _fin():
        v8_ref[...] = v8_scr[...]
        i8_ref[...] = i8_scr[...]
        s_ref[...] = s_scr[...]


@functools.partial(jax.jit, static_argnames=())
def _decode_step(emb, h, W_ih, W_hh, W_out):
    grid = (NT,)
    out = pl.pallas_call(
        _step_kernel,
        grid=grid,
        in_specs=[
            pl.BlockSpec((BK, HIDDEN), lambda j: (0, 0)),
            pl.BlockSpec((BK, HIDDEN), lambda j: (0, 0)),
            pl.BlockSpec((HIDDEN, HIDDEN), lambda j: (0, 0)),
            pl.BlockSpec((HIDDEN, HIDDEN), lambda j: (0, 0)),
            pl.BlockSpec((HIDDEN, TILE), lambda j: (0, j)),
        ],
        out_specs=[
            pl.BlockSpec((BK, HIDDEN), lambda j: (0, 0)),
            pl.BlockSpec((BK, 8), lambda j: (0, 0)),
            pl.BlockSpec((BK, 8), lambda j: (0, 0)),
            pl.BlockSpec((BK, 8), lambda j: (0, 0)),
        ],
        out_shape=[
            jax.ShapeDtypeStruct((BK, HIDDEN), jnp.float32),
            jax.ShapeDtypeStruct((BK, 8), jnp.float32),
            jax.ShapeDtypeStruct((BK, 8), jnp.int32),
            jax.ShapeDtypeStruct((BK, 8), jnp.float32),
        ],
        scratch_shapes=[
            pltpu.VMEM((BK, HIDDEN), jnp.float32),
            pltpu.VMEM((BK, 8), jnp.float32),
            pltpu.VMEM((BK, 8), jnp.int32),
            pltpu.VMEM((BK, 8), jnp.float32),
        ],
    )(emb, h, W_ih, W_hh, W_out)
    return out


def kernel(dec_hidden, embedding, W_ih, W_hh, W_out):
    pos_index = (jnp.arange(BATCH) * BEAM).reshape(-1, 1)
    h = jnp.tile(dec_hidden[0], (BEAM, 1))
    seq_scores = jnp.full((BK,), NEG_INF, dtype=jnp.float32)
    seq_scores = seq_scores.at[jnp.arange(BATCH) * BEAM].set(0.0)
    dec_input = jnp.full((BK,), SOS_ID, dtype=jnp.int32)

    stored_scores = []
    stored_predecessors = []
    stored_symbols = []

    for _ in range(MAX_LEN):
        emb = _sc_gather(embedding, dec_input)
        h_new, v8, i8, s = _decode_step(emb, h, W_ih, W_hh, W_out)
        m = jnp.max(v8, axis=1, keepdims=True)
        lse = jnp.log(s[:, 0:1] * jnp.exp(-m))
        logp8 = (v8 - m) - lse
        cand = seq_scores[:, None] + logp8
        candv = cand.reshape(BATCH, BEAM * 8)
        beam_local = jnp.broadcast_to(
            (jnp.arange(BK) % BEAM)[:, None], (BK, 8))
        key = (beam_local * VOCAB + i8).reshape(BATCH, BEAM * 8)
        negv, skey = lax.sort((-candv, key), dimension=1, num_keys=2)
        sc = -negv[:, :BEAM]
        skey8 = skey[:, :BEAM]
        symb = (skey8 % VOCAB).reshape(BK)
        beam_of = skey8 // VOCAB
        flat = (jnp.arange(BATCH)[:, None] * BEAM + beam_of).reshape(BK)
        dec_input = symb.astype(jnp.int32)
        seq_scores = sc.reshape(BK)
        predecessors = flat
        h = jnp.take(h_new, predecessors, axis=0)
        stored_scores.append(seq_scores)
        seq_scores = jnp.where(dec_input == EOS_ID, NEG_INF, seq_scores)
        stored_predecessors.append(predecessors)
        stored_symbols.append(dec_input)

    sorted_score, sorted_idx = lax.top_k(
        stored_scores[-1].reshape(BATCH, BEAM), BEAM)
    t_pred = (sorted_idx + pos_index).reshape(BK)
    seq_rev = []
    for t in range(MAX_LEN - 1, -1, -1):
        seq_rev.append(jnp.take(stored_symbols[t], t_pred, axis=0))
        t_pred = jnp.take(stored_predecessors[t], t_pred, axis=0)
    topk_sequence = jnp.stack(seq_rev[::-1], axis=0).T.reshape(
        BATCH, BEAM, MAX_LEN)
    return sorted_score, topk_sequence


# speedup vs baseline: 1.0076x; 1.0076x over previous
import functools
import jax
import jax.numpy as jnp
from jax import lax
from jax.experimental import pallas as pl
from jax.experimental.pallas import tpu as pltpu
from jax.experimental.pallas import tpu_sc as plsc

BATCH = 16
BEAM = 8
VOCAB = 100000
HIDDEN = 512
MAX_LEN = 8
SOS_ID = 1
EOS_ID = 2
NEG_INF = -1e9
BK = BATCH * BEAM

TILE = 4096
NT = (VOCAB + TILE - 1) // TILE
BIG_NEG = -1e30


_N_WORKERS = 16
_ROWS_PER_W = BK // _N_WORKERS


def _sc_gather_body(table_hbm, idx_hbm, out_hbm, idx_v, rows_v, sem):
    wid = lax.axis_index("s") * 2 + lax.axis_index("c")

    @pl.when(wid < _N_WORKERS)
    def _():
        base = wid * _ROWS_PER_W
        pltpu.sync_copy(idx_hbm.at[pl.ds(base, _ROWS_PER_W)], idx_v)
        pltpu.async_copy(table_hbm.at[idx_v], rows_v, sem).wait()
        pltpu.sync_copy(rows_v, out_hbm.at[pl.ds(base, _ROWS_PER_W)])


def _sc_gather(embedding, idx):
    mesh = plsc.VectorSubcoreMesh(core_axis_name="c", subcore_axis_name="s")
    k = functools.partial(
        pl.kernel, mesh=mesh,
        out_type=jax.ShapeDtypeStruct((BK, HIDDEN), jnp.float32),
        scratch_types=[
            pltpu.VMEM((_ROWS_PER_W,), jnp.int32),
            pltpu.VMEM((_ROWS_PER_W, HIDDEN), jnp.float32),
            pltpu.SemaphoreType.DMA,
        ],
    )(_sc_gather_body)
    return k(embedding, idx)


def _step_kernel(emb_ref, hprev_ref, wih_ref, whh_ref, wout_ref,
                 hout_ref, v8_ref, i8_ref, s_ref,
                 h_scr, v8_scr, i8_scr, s_scr):
    j = pl.program_id(0)

    @pl.when(j == 0)
    def _init():
        emb = emb_ref[...]
        h = jnp.tanh(
            jax.lax.dot_general(emb, wih_ref[...], (((1,), (0,)), ((), ())),
                                preferred_element_type=jnp.float32)
            + jax.lax.dot_general(hprev_ref[...], whh_ref[...],
                                  (((1,), (0,)), ((), ())),
                                  preferred_element_type=jnp.float32))
        h_scr[...] = h
        hout_ref[...] = h
        v8_scr[...] = jnp.full((BK, 8), BIG_NEG, dtype=jnp.float32)
        i8_scr[...] = jnp.zeros((BK, 8), dtype=jnp.int32)
        s_scr[...] = jnp.zeros((BK, 8), dtype=jnp.float32)

    h = h_scr[...]
    logits = jax.lax.dot_general(h, wout_ref[...], (((1,), (0,)), ((), ())),
                                 preferred_element_type=jnp.float32)
    col = jax.lax.broadcasted_iota(jnp.int32, (BK, TILE), 1)
    logits = jnp.where(col < VOCAB - j * TILE, logits, BIG_NEG)
    s = s_scr[:, 0:1] + jnp.sum(jnp.exp(logits), axis=1, keepdims=True)
    s_scr[...] = jnp.broadcast_to(s, (BK, 8))
    tmax = jnp.max(logits, axis=1, keepdims=True)

    v8 = v8_scr[...]
    i8 = i8_scr[...]
    slot_iota = jax.lax.broadcasted_iota(jnp.int32, (BK, 8), 1)

    def cond(carry):
        L, m, v8c, i8c, k = carry
        return jnp.logical_and(
            k < 8, jnp.any(m > jnp.min(v8c, axis=1, keepdims=True)))

    def body(carry):
        L, m, v8c, i8c, k = carry
        pos = jnp.min(jnp.where(L == m, col, TILE), axis=1, keepdims=True)
        minv = jnp.min(v8c, axis=1, keepdims=True)
        improve = m > minv
        is_min = v8c == minv
        si = jnp.min(jnp.where(is_min, slot_iota, 8), axis=1, keepdims=True)
        repl = jnp.logical_and(slot_iota == si, improve)
        v8c = jnp.where(repl, jnp.broadcast_to(m, (BK, 8)), v8c)
        i8c = jnp.where(repl, jnp.broadcast_to(pos + j * TILE, (BK, 8)), i8c)
        L = jnp.where(jnp.logical_and(col == pos, improve), BIG_NEG, L)
        m2 = jnp.max(L, axis=1, keepdims=True)
        return (L, m2, v8c, i8c, k + 1)

    init = (logits, tmax, v8, i8, jnp.int32(0))
    _, _, v8, i8, _ = lax.while_loop(cond, body, init)
    v8_scr[...] = v8
    i8_scr[...] = i8

    @pl.when(j == NT - 1)
    def _fin():
        v8_ref[...] = v8_scr[...]
        i8_ref[...] = i8_scr[...]
        s_ref[...] = s_scr[...]


@functools.partial(jax.jit, static_argnames=())
def _decode_step(emb, h, W_ih, W_hh, W_out):
    grid = (NT,)
    out = pl.pallas_call(
        _step_kernel,
        grid=grid,
        in_specs=[
            pl.BlockSpec((BK, HIDDEN), lambda j: (0, 0)),
            pl.BlockSpec((BK, HIDDEN), lambda j: (0, 0)),
            pl.BlockSpec((HIDDEN, HIDDEN), lambda j: (0, 0)),
            pl.BlockSpec((HIDDEN, HIDDEN), lambda j: (0, 0)),
            pl.BlockSpec((HIDDEN, TILE), lambda j: (0, j)),
        ],
        out_specs=[
            pl.BlockSpec((BK, HIDDEN), lambda j: (0, 0)),
            pl.BlockSpec((BK, 8), lambda j: (0, 0)),
            pl.BlockSpec((BK, 8), lambda j: (0, 0)),
            pl.BlockSpec((BK, 8), lambda j: (0, 0)),
        ],
        out_shape=[
            jax.ShapeDtypeStruct((BK, HIDDEN), jnp.float32),
            jax.ShapeDtypeStruct((BK, 8), jnp.float32),
            jax.ShapeDtypeStruct((BK, 8), jnp.int32),
            jax.ShapeDtypeStruct((BK, 8), jnp.float32),
        ],
        scratch_shapes=[
            pltpu.VMEM((BK, HIDDEN), jnp.float32),
            pltpu.VMEM((BK, 8), jnp.float32),
            pltpu.VMEM((BK, 8), jnp.int32),
            pltpu.VMEM((BK, 8), jnp.float32),
        ],
    )(emb, h, W_ih, W_hh, W_out)
    return out


def kernel(dec_hidden, embedding, W_ih, W_hh, W_out):
    pos_index = (jnp.arange(BATCH) * BEAM).reshape(-1, 1)
    h = jnp.tile(dec_hidden[0], (BEAM, 1))
    seq_scores = jnp.full((BK,), NEG_INF, dtype=jnp.float32)
    seq_scores = seq_scores.at[jnp.arange(BATCH) * BEAM].set(0.0)
    dec_input = jnp.full((BK,), SOS_ID, dtype=jnp.int32)

    stored_scores = []
    stored_predecessors = []
    stored_symbols = []

    for _ in range(MAX_LEN):
        emb = _sc_gather(embedding, dec_input)
        h_new, v8, i8, s = _decode_step(emb, h, W_ih, W_hh, W_out)
        m = jnp.max(v8, axis=1, keepdims=True)
        lse = jnp.log(s[:, 0:1] * jnp.exp(-m))
        logp8 = (v8 - m) - lse
        cand = seq_scores[:, None] + logp8
        candv = cand.reshape(BATCH, BEAM * 8)
        beam_local = jnp.broadcast_to(
            (jnp.arange(BK) % BEAM)[:, None], (BK, 8))
        key = (beam_local * VOCAB + i8).reshape(BATCH, BEAM * 8)
        negv, skey = lax.sort((-candv, key), dimension=1, num_keys=2)
        sc = -negv[:, :BEAM]
        skey8 = skey[:, :BEAM]
        symb = (skey8 % VOCAB).reshape(BK)
        beam_of = skey8 // VOCAB
        flat = (jnp.arange(BATCH)[:, None] * BEAM + beam_of).reshape(BK)
        dec_input = symb.astype(jnp.int32)
        seq_scores = sc.reshape(BK)
        predecessors = flat
        h = jnp.take(h_new, predecessors, axis=0)
        stored_scores.append(seq_scores)
        seq_scores = jnp.where(dec_input == EOS_ID, NEG_INF, seq_scores)
        stored_predecessors.append(predecessors)
        stored_symbols.append(dec_input)

    sorted_score, sorted_idx = lax.top_k(
        stored_scores[-1].reshape(BATCH, BEAM), BEAM)
    t_pred = (sorted_idx + pos_index).reshape(BK)
    seq_rev = []
    for t in range(MAX_LEN - 1, -1, -1):
        seq_rev.append(jnp.take(stored_symbols[t], t_pred, axis=0))
        t_pred = jnp.take(stored_predecessors[t], t_pred, axis=0)
    topk_sequence = jnp.stack(seq_rev[::-1], axis=0).T.reshape(
        BATCH, BEAM, MAX_LEN)
    return sorted_score, topk_sequence
